# baseline (speedup 1.0000x reference)
import jax
import jax.numpy as jnp
from jax import lax
from jax.experimental import pallas as pl
from jax.experimental.pallas import tpu as pltpu
from jax.experimental.pallas import tpu_sc as plsc

_B, _T, _D = 4, 4096, 2048
_NC, _NS, _L = 2, 16, 16
_NW = _NC * _NS
_TPW = _T // _NW
_C = 8
_CHUNK = _C * _D
_NCHUNK = _TPW // _C


def _sc_body(x_hbm, pos_hbm, out_hbm, pos_buf, work_buf):
    wid = lax.axis_index("s") * _NC + lax.axis_index("c")
    t0 = wid * _TPW

    def tchunk_body(tc, _):
        base_t = t0 + tc * _C
        pltpu.sync_copy(pos_hbm.at[pl.ds(base_t * _D, _CHUNK)], pos_buf)

        def batch_body(b, _):
            off = (b * _T + base_t) * _D
            pltpu.sync_copy(x_hbm.at[pl.ds(off, _CHUNK)], work_buf)

            @plsc.parallel_loop(0, _CHUNK, step=_L, unroll=8)
            def _(j):
                plsc.addupdate(work_buf.at[pl.ds(j, _L)], pos_buf[pl.ds(j, _L)])

            pltpu.sync_copy(work_buf, out_hbm.at[pl.ds(off, _CHUNK)])
            return 0

        lax.fori_loop(0, _B, batch_body, 0)
        return 0

    lax.fori_loop(0, _NCHUNK, tchunk_body, 0)


def kernel(x, pos_table):
    b, t, d = x.shape
    mesh = plsc.VectorSubcoreMesh(core_axis_name="c", subcore_axis_name="s")
    out = pl.kernel(
        _sc_body,
        out_type=jax.ShapeDtypeStruct((b * t * d,), x.dtype),
        mesh=mesh,
        scratch_types=[
            pltpu.VMEM((_CHUNK,), jnp.float32),
            pltpu.VMEM((_CHUNK,), jnp.float32),
        ],
    )(x.reshape(-1), pos_table.reshape(-1))
    return out.reshape(b, t, d)


# speedup vs baseline: 1.3115x; 1.3115x over previous
import jax
import jax.numpy as jnp
from jax import lax
from jax.experimental import pallas as pl
from jax.experimental.pallas import tpu as pltpu
from jax.experimental.pallas import tpu_sc as plsc

_B, _T, _D = 4, 4096, 2048
_NC, _NS, _L = 2, 16, 16
_NW = _NC * _NS
_TPW = _T // _NW
_C = 8
_CHUNK = _C * _D
_NCHUNK = _TPW // _C
_S = _NCHUNK * _B


def _sc_body(x_hbm, pos_hbm, out_hbm,
             in0, in1, ou0, ou1, po0, po1,
             si0, si1, so0, so1, sp0, sp1):
    ins, ous, pos = (in0, in1), (ou0, ou1), (po0, po1)
    sin, son, spo = (si0, si1), (so0, so1), (sp0, sp1)

    wid = lax.axis_index("s") * _NC + lax.axis_index("c")
    t0 = wid * _TPW

    def x_slice(s):
        tc = s // _B
        b = s % _B
        return pl.ds((b * _T + t0 + tc * _C) * _D, _CHUNK)

    def pos_slice(tc):
        return pl.ds((t0 + tc * _C) * _D, _CHUNK)

    pltpu.async_copy(x_hbm.at[x_slice(0)], in0, si0)
    pltpu.async_copy(x_hbm.at[x_slice(1)], in1, si1)
    pltpu.async_copy(pos_hbm.at[pos_slice(0)], po0, sp0)
    pltpu.async_copy(pos_hbm.at[pos_slice(1)], po1, sp1)

    def outer(g, _):
        for tc_k in (0, 1):
            tc = 2 * g + tc_k
            pltpu.make_async_copy(
                pos_hbm.at[pos_slice(tc)], pos[tc_k], spo[tc_k]).wait()
            for b_k in range(_B):
                k = b_k % 2
                s = tc * _B + b_k
                pltpu.make_async_copy(
                    x_hbm.at[x_slice(s)], ins[k], sin[k]).wait()
                @pl.when(s >= 2)
                def _():
                    pltpu.make_async_copy(
                        ous[k], out_hbm.at[x_slice(s - 2)], son[k]).wait()

                @plsc.parallel_loop(0, _CHUNK, step=_L, unroll=8)
                def _(j):
                    ous[k][pl.ds(j, _L)] = (
                        ins[k][pl.ds(j, _L)] + pos[tc_k][pl.ds(j, _L)])

                pltpu.async_copy(ous[k], out_hbm.at[x_slice(s)], son[k])

                @pl.when(s + 2 < _S)
                def _():
                    pltpu.async_copy(x_hbm.at[x_slice(s + 2)], ins[k], sin[k])

                if b_k == _B - 1:
                    @pl.when(tc + 2 < _NCHUNK)
                    def _():
                        pltpu.async_copy(
                            pos_hbm.at[pos_slice(tc + 2)], pos[tc_k], spo[tc_k])
        return 0

    lax.fori_loop(0, _NCHUNK // 2, outer, 0)

    pltpu.make_async_copy(ous[0], out_hbm.at[x_slice(_S - 2)], son[0]).wait()
    pltpu.make_async_copy(ous[1], out_hbm.at[x_slice(_S - 1)], son[1]).wait()


def kernel(x, pos_table):
    b, t, d = x.shape
    mesh = plsc.VectorSubcoreMesh(core_axis_name="c", subcore_axis_name="s")
    out = pl.kernel(
        _sc_body,
        out_type=jax.ShapeDtypeStruct((b * t * d,), x.dtype),
        mesh=mesh,
        scratch_types=[
            pltpu.VMEM((_CHUNK,), jnp.float32),
            pltpu.VMEM((_CHUNK,), jnp.float32),
            pltpu.VMEM((_CHUNK,), jnp.float32),
            pltpu.VMEM((_CHUNK,), jnp.float32),
            pltpu.VMEM((_CHUNK,), jnp.float32),
            pltpu.VMEM((_CHUNK,), jnp.float32),
            pltpu.SemaphoreType.DMA,
            pltpu.SemaphoreType.DMA,
            pltpu.SemaphoreType.DMA,
            pltpu.SemaphoreType.DMA,
            pltpu.SemaphoreType.DMA,
            pltpu.SemaphoreType.DMA,
        ],
    )(x.reshape(-1), pos_table.reshape(-1))
    return out.reshape(b, t, d)


# speedup vs baseline: 1.3313x; 1.0151x over previous
import jax
import jax.numpy as jnp
from jax import lax
from jax.experimental import pallas as pl
from jax.experimental.pallas import tpu as pltpu
from jax.experimental.pallas import tpu_sc as plsc

_B, _T, _D = 4, 4096, 2048
_NC, _NS, _L = 2, 16, 16
_NW = _NC * _NS
_TPW = _T // _NW
_C = 8
_CHUNK = _C * _D
_NCHUNK = _TPW // _C
_S = _NCHUNK * _B
_NSLOT = 4


def _sc_body(x_hbm, pos_hbm, out_hbm,
             w0, w1, w2, w3, po0, po1,
             sw0, sw1, sw2, sw3, so0, so1, so2, so3, sp0, sp1):
    wk = (w0, w1, w2, w3)
    swi = (sw0, sw1, sw2, sw3)
    swo = (so0, so1, so2, so3)
    pos, spo = (po0, po1), (sp0, sp1)

    wid = lax.axis_index("s") * _NC + lax.axis_index("c")
    t0 = wid * _TPW

    def x_slice(s):
        tc = s // _B
        b = s % _B
        return pl.ds((b * _T + t0 + tc * _C) * _D, _CHUNK)

    def pos_slice(tc):
        return pl.ds((t0 + tc * _C) * _D, _CHUNK)

    pltpu.async_copy(x_hbm.at[x_slice(0)], wk[0], swi[0])
    pltpu.async_copy(x_hbm.at[x_slice(1)], wk[1], swi[1])
    pltpu.async_copy(x_hbm.at[x_slice(2)], wk[2], swi[2])
    pltpu.async_copy(pos_hbm.at[pos_slice(0)], po0, sp0)
    pltpu.async_copy(pos_hbm.at[pos_slice(1)], po1, sp1)

    def outer(g, _):
        for tc_k in (0, 1):
            tc = 2 * g + tc_k
            pltpu.make_async_copy(
                pos_hbm.at[pos_slice(tc)], pos[tc_k], spo[tc_k]).wait()
            for b_k in range(_B):
                s = tc * _B + b_k
                k = b_k
                kn = (b_k + 3) % _NSLOT
                pltpu.make_async_copy(
                    x_hbm.at[x_slice(s)], wk[k], swi[k]).wait()

                @plsc.parallel_loop(0, _CHUNK, step=_L, unroll=16)
                def _(j):
                    plsc.addupdate(wk[k].at[pl.ds(j, _L)],
                                   pos[tc_k][pl.ds(j, _L)])

                pltpu.async_copy(wk[k], out_hbm.at[x_slice(s)], swo[k])

                @pl.when(s + 3 < _S)
                def _():
                    @pl.when(s >= 1)
                    def _():
                        pltpu.make_async_copy(
                            wk[kn], out_hbm.at[x_slice(s - 1)], swo[kn]).wait()
                    pltpu.async_copy(x_hbm.at[x_slice(s + 3)], wk[kn], swi[kn])

                if b_k == _B - 1:
                    @pl.when(tc + 2 < _NCHUNK)
                    def _():
                        pltpu.async_copy(
                            pos_hbm.at[pos_slice(tc + 2)], pos[tc_k], spo[tc_k])
        return 0

    lax.fori_loop(0, _NCHUNK // 2, outer, 0)

    for s in range(_S - 4, _S):
        k = s % _NSLOT
        pltpu.make_async_copy(wk[k], out_hbm.at[x_slice(s)], swo[k]).wait()


def kernel(x, pos_table):
    b, t, d = x.shape
    mesh = plsc.VectorSubcoreMesh(core_axis_name="c", subcore_axis_name="s")
    out = pl.kernel(
        _sc_body,
        out_type=jax.ShapeDtypeStruct((b * t * d,), x.dtype),
        mesh=mesh,
        scratch_types=(
            [pltpu.VMEM((_CHUNK,), jnp.float32)] * 6
            + [pltpu.SemaphoreType.DMA] * 10
        ),
    )(x.reshape(-1), pos_table.reshape(-1))
    return out.reshape(b, t, d)


# speedup vs baseline: 5.4953x; 4.1277x over previous
import jax
import jax.numpy as jnp
from jax.experimental import pallas as pl
from jax.experimental.pallas import tpu as pltpu


_TB = 1024


def _add_kernel(x_ref, pe_ref, o_ref):
    o_ref[...] = x_ref[...] + pe_ref[...]


def kernel(x, pos_table):
    b, t, d = x.shape
    grid = (t // _TB, b)
    return pl.pallas_call(
        _add_kernel,
        grid=grid,
        in_specs=[
            pl.BlockSpec((1, _TB, d), lambda i, j: (j, i, 0)),
            pl.BlockSpec((_TB, d), lambda i, j: (i, 0)),
        ],
        out_specs=pl.BlockSpec((1, _TB, d), lambda i, j: (j, i, 0)),
        out_shape=jax.ShapeDtypeStruct((b, t, d), x.dtype),
    )(x, pos_table)


# speedup vs baseline: 5.4976x; 1.0004x over previous
import jax
import jax.numpy as jnp
from jax.experimental import pallas as pl


_TB = 1024


def _add_kernel(x_ref, pe_ref, o_ref):
    o_ref[...] = x_ref[...] + pe_ref[...]


def kernel(x, pos_table):
    b, t, d = x.shape
    nt = t // _TB
    out = pl.pallas_call(
        _add_kernel,
        grid=(nt, b),
        in_specs=[
            pl.BlockSpec((_TB, d), lambda i, j: (j * nt + i, 0)),
            pl.BlockSpec((_TB, d), lambda i, j: (i, 0)),
        ],
        out_specs=pl.BlockSpec((_TB, d), lambda i, j: (j * nt + i, 0)),
        out_shape=jax.ShapeDtypeStruct((b * t, d), x.dtype),
    )(x.reshape(b * t, d), pos_table)
    return out.reshape(b, t, d)


# speedup vs baseline: 5.4986x; 1.0002x over previous
import jax
import jax.numpy as jnp
from jax.experimental import pallas as pl


_TB = 1024


def _add_kernel(x_ref, pe_ref, o_ref):
    o_ref[...] = x_ref[...] + pe_ref[...]


def kernel(x, pos_table):
    b, t, d = x.shape
    grid = (t // _TB, b)
    return pl.pallas_call(
        _add_kernel,
        grid=grid,
        in_specs=[
            pl.BlockSpec((1, _TB, d), lambda i, j: (j, i, 0)),
            pl.BlockSpec((_TB, d), lambda i, j: (i, 0)),
        ],
        out_specs=pl.BlockSpec((1, _TB, d), lambda i, j: (j, i, 0)),
        out_shape=jax.ShapeDtypeStruct((b, t, d), x.dtype),
    )(x, pos_table)
